# baseline (speedup 1.0000x reference)
import functools

import jax
import jax.numpy as jnp
from jax import lax
from jax.experimental import pallas as pl
from jax.experimental.pallas import tpu as pltpu
from jax.experimental.pallas import tpu_sc as plsc

B = 16384
K = 16
CHUNK = 128
RPR = 8


@jax.jit
def _bpr_sc(user, pos_item, neg_item, embedding_user, embedding_item):
    info = plsc.get_sparse_core_info()
    NC, NS = info.num_cores, info.num_subcores
    NW = NC * NS
    b_per_w = B // NW
    n_chunks = b_per_w // CHUNK
    n_user = embedding_user.shape[0]
    n_item = embedding_item.shape[0]

    mesh = plsc.VectorSubcoreMesh(core_axis_name="c", subcore_axis_name="s")

    @functools.partial(
        pl.kernel,
        mesh=mesh,
        compiler_params=pltpu.CompilerParams(
            needs_layout_passes=False, use_tc_tiling_on_sc=True,
            disable_bounds_checks=True),
        out_type=jax.ShapeDtypeStruct((NW, n_chunks, CHUNK), jnp.float32),
        scratch_types=[
            pltpu.VMEM((n_chunks, CHUNK), jnp.int32),
            pltpu.VMEM((n_chunks, CHUNK), jnp.int32),
            pltpu.VMEM((n_chunks, CHUNK), jnp.int32),
            pltpu.VMEM((CHUNK,), jnp.int32),
            pltpu.VMEM((CHUNK,), jnp.int32),
            pltpu.VMEM((CHUNK,), jnp.int32),
            pltpu.VMEM((CHUNK, CHUNK), jnp.float32),
            pltpu.VMEM((CHUNK, CHUNK), jnp.float32),
            pltpu.VMEM((CHUNK, CHUNK), jnp.float32),
            pltpu.VMEM((CHUNK, K + 1), jnp.float32),
            pltpu.VMEM((n_chunks, CHUNK), jnp.float32),
            pltpu.SemaphoreType.DMA,
        ],
    )
    def k(user_hbm, pos_hbm, neg_hbm, eu_hbm, ei_hbm, out_hbm,
          uidx_v, pidx_v, nidx_v, utj_v, ptj_v, ntj_v,
          u_g, p_g, n_g, d_pad, out_v, sem):
        wid = lax.axis_index("s") * NC + lax.axis_index("c")
        pltpu.sync_copy(user_hbm.at[wid], uidx_v)
        pltpu.sync_copy(pos_hbm.at[wid], pidx_v)
        pltpu.sync_copy(neg_hbm.at[wid], nidx_v)
        lane = lax.iota(jnp.int32, K)
        for c in range(n_chunks):
            for src, dst in ((uidx_v, utj_v), (pidx_v, ptj_v),
                             (nidx_v, ntj_v)):
                def tj_body(v, _, src=src, dst=dst, c=c):
                    sl = pl.ds(v * K, K)
                    dst[sl] = src[c, sl] >> 3
                    return 0
                lax.fori_loop(0, CHUNK // K, tj_body, 0)
            cps = [
                pltpu.async_copy(eu_hbm.at[utj_v], u_g, sem),
                pltpu.async_copy(ei_hbm.at[ptj_v], p_g, sem),
                pltpu.async_copy(ei_hbm.at[ntj_v], n_g, sem),
            ]
            for cp in cps:
                cp.wait()

            def dbody(g, _, c=c):
                base = g * K
                uv = uidx_v[c, pl.ds(base, K)] & 7
                pv = pidx_v[c, pl.ds(base, K)] & 7
                nv = nidx_v[c, pl.ds(base, K)] & 7
                for j in range(K):
                    i = base + j
                    u = u_g[i, pl.ds(uv[j] * K, K)]
                    p = p_g[i, pl.ds(pv[j] * K, K)]
                    n = n_g[i, pl.ds(nv[j] * K, K)]
                    d_pad[i, pl.ds(0, K)] = u * (p - n)
                return 0

            lax.fori_loop(0, CHUNK // K, dbody, 0)

            def gbody(g, _, c=c):
                rows = g * K + lane
                acc = plsc.load_gather(d_pad, [rows, jnp.zeros((K,), jnp.int32)])
                for kk in range(1, K):
                    acc = acc + plsc.load_gather(
                        d_pad, [rows, jnp.full((K,), kk, jnp.int32)])
                out_v[c, pl.ds(g * K, K)] = acc
                return 0

            lax.fori_loop(0, CHUNK // K, gbody, 0)
        pltpu.sync_copy(out_v, out_hbm.at[wid])

    out = k(
        user.reshape(NW, n_chunks, CHUNK),
        pos_item.reshape(NW, n_chunks, CHUNK),
        neg_item.reshape(NW, n_chunks, CHUNK),
        embedding_user.reshape(n_user // RPR, K * RPR),
        embedding_item.reshape(n_item // RPR, K * RPR),
    )
    return out.reshape(B)


def kernel(user, pos_item, neg_item, embedding_user, embedding_item):
    return _bpr_sc(user, pos_item, neg_item, embedding_user, embedding_item)


# speedup vs baseline: 1.0189x; 1.0189x over previous
import functools

import jax
import jax.numpy as jnp
from jax import lax
from jax.experimental import pallas as pl
from jax.experimental.pallas import tpu as pltpu
from jax.experimental.pallas import tpu_sc as plsc

B = 16384
K = 16
CHUNK = 128


@jax.jit
def _bpr_sc(user, pos_item, neg_item, embedding_user, embedding_item):
    info = plsc.get_sparse_core_info()
    NC, NS = info.num_cores, info.num_subcores
    NW = NC * NS
    b_per_w = B // NW
    n_chunks = b_per_w // CHUNK

    mesh = plsc.VectorSubcoreMesh(core_axis_name="c", subcore_axis_name="s")

    @functools.partial(
        pl.kernel,
        mesh=mesh,
        compiler_params=pltpu.CompilerParams(
            needs_layout_passes=False, use_tc_tiling_on_sc=False,
            disable_bounds_checks=True),
        out_type=jax.ShapeDtypeStruct((NW, n_chunks, CHUNK), jnp.float32),
        scratch_types=[
            pltpu.VMEM((n_chunks, CHUNK), jnp.int32),
            pltpu.VMEM((n_chunks, CHUNK), jnp.int32),
            pltpu.VMEM((n_chunks, CHUNK), jnp.int32),
            pltpu.VMEM((CHUNK, K), jnp.float32),
            pltpu.VMEM((CHUNK, K), jnp.float32),
            pltpu.VMEM((CHUNK, K), jnp.float32),
            pltpu.VMEM((CHUNK, K + 1), jnp.float32),
            pltpu.VMEM((n_chunks, CHUNK), jnp.float32),
            pltpu.SemaphoreType.DMA,
        ],
    )
    def k(user_hbm, pos_hbm, neg_hbm, eu_hbm, ei_hbm, out_hbm,
          uidx_v, pidx_v, nidx_v, u_v, p_v, n_v, d_pad, out_v, sem):
        wid = lax.axis_index("s") * NC + lax.axis_index("c")
        pltpu.sync_copy(user_hbm.at[wid], uidx_v)
        pltpu.sync_copy(pos_hbm.at[wid], pidx_v)
        pltpu.sync_copy(neg_hbm.at[wid], nidx_v)
        lane = lax.iota(jnp.int32, K)
        for c in range(n_chunks):
            cps = [
                pltpu.async_copy(eu_hbm.at[uidx_v.at[c]], u_v, sem),
                pltpu.async_copy(ei_hbm.at[pidx_v.at[c]], p_v, sem),
                pltpu.async_copy(ei_hbm.at[nidx_v.at[c]], n_v, sem),
            ]
            for cp in cps:
                cp.wait()

            def dbody(i, _):
                d_pad[i, pl.ds(0, K)] = u_v[i] * (p_v[i] - n_v[i])
                return 0

            lax.fori_loop(0, CHUNK, dbody, 0)

            def gbody(g, _, c=c):
                rows = g * K + lane
                acc = plsc.load_gather(d_pad, [rows, jnp.zeros((K,), jnp.int32)])
                for kk in range(1, K):
                    acc = acc + plsc.load_gather(
                        d_pad, [rows, jnp.full((K,), kk, jnp.int32)])
                out_v[c, pl.ds(g * K, K)] = acc
                return 0

            lax.fori_loop(0, CHUNK // K, gbody, 0)
        pltpu.sync_copy(out_v, out_hbm.at[wid])

    out = k(
        user.reshape(NW, n_chunks, CHUNK),
        pos_item.reshape(NW, n_chunks, CHUNK),
        neg_item.reshape(NW, n_chunks, CHUNK),
        embedding_user,
        embedding_item,
    )
    return out.reshape(B)


def kernel(user, pos_item, neg_item, embedding_user, embedding_item):
    return _bpr_sc(user, pos_item, neg_item, embedding_user, embedding_item)
